# Initial kernel scaffold; baseline (speedup 1.0000x reference)
import functools

import jax
import jax.numpy as jnp
from jax import lax
from jax.experimental import pallas as pl
from jax.experimental.pallas import tpu as pltpu
from jax.experimental.pallas import tpu_sc as plsc

N = 10000
E = 320000
IN_DIM = 128
HID = 128
H0 = 8
H1 = 1
CLS = 64
NEG = 0.2

NP = 10240
NSC = 2
NTL = 16
LANES = 16

_MESH = plsc.VectorSubcoreMesh(core_axis_name="c", subcore_axis_name="s")
_SC_PARAMS = pltpu.CompilerParams(needs_layout_passes=False)


def _tc1_body(x_ref, w_ref, al_ref, ar_ref, feat_ref, eler_ref):
  x = x_ref[...]
  p = jnp.dot(x, w_ref[...], preferred_element_type=jnp.float32)
  feat_ref[...] = p
  fr = p.reshape(p.shape[0], H0, HID)
  el = jnp.sum(fr * al_ref[...][None], axis=-1)
  er = jnp.sum(fr * ar_ref[...][None], axis=-1)
  eler_ref[...] = jnp.concatenate([el, er], axis=1)


def _tc1(x_pad, W0, al0, ar0):
  blk = 512
  grid = (NP // blk,)
  return pl.pallas_call(
      _tc1_body,
      grid=grid,
      in_specs=[
          pl.BlockSpec((blk, IN_DIM), lambda i: (i, 0)),
          pl.BlockSpec((IN_DIM, H0 * HID), lambda i: (0, 0)),
          pl.BlockSpec((H0, HID), lambda i: (0, 0)),
          pl.BlockSpec((H0, HID), lambda i: (0, 0)),
      ],
      out_specs=[
          pl.BlockSpec((blk, H0 * HID), lambda i: (i, 0)),
          pl.BlockSpec((blk, 2 * H0), lambda i: (i, 0)),
      ],
      out_shape=[
          jax.ShapeDtypeStruct((NP, H0 * HID), jnp.float32),
          jax.ShapeDtypeStruct((NP, 2 * H0), jnp.float32),
      ],
  )(x_pad, W0, al0, ar0)


def _tc2_body(h_ref, w_ref, al_ref, ar_ref, feat_ref, res_ref, eler_ref):
  x = jnp.maximum(h_ref[...], 0.0)
  p = jnp.dot(x, w_ref[...], preferred_element_type=jnp.float32)
  f = p[:, :CLS]
  feat_ref[...] = f
  res_ref[...] = p[:, CLS:]
  el = jnp.sum(f * al_ref[...], axis=-1, keepdims=True)
  er = jnp.sum(f * ar_ref[...], axis=-1, keepdims=True)
  eler_ref[...] = jnp.concatenate([el, er], axis=1)


def _tc2(h_pad, Wcat, al1, ar1):
  blk = 512
  grid = (NP // blk,)
  return pl.pallas_call(
      _tc2_body,
      grid=grid,
      in_specs=[
          pl.BlockSpec((blk, H0 * HID), lambda i: (i, 0)),
          pl.BlockSpec((H0 * HID, 2 * CLS), lambda i: (0, 0)),
          pl.BlockSpec((1, CLS), lambda i: (0, 0)),
          pl.BlockSpec((1, CLS), lambda i: (0, 0)),
      ],
      out_specs=[
          pl.BlockSpec((blk, CLS), lambda i: (i, 0)),
          pl.BlockSpec((blk, CLS), lambda i: (i, 0)),
          pl.BlockSpec((blk, 2), lambda i: (i, 0)),
      ],
      out_shape=[
          jax.ShapeDtypeStruct((NP, CLS), jnp.float32),
          jax.ShapeDtypeStruct((NP, CLS), jnp.float32),
          jax.ShapeDtypeStruct((NP, 2), jnp.float32),
      ],
  )(h_pad, Wcat, al1, ar1)


def _make_attn(H):
  HPS = max(H // NSC, 1)
  EP = E // NTL
  VE = EP // LANES
  TS = NP // NTL
  NV = NP // LANES

  def body(src_hbm, dst_hbm, eler_hbm, out_hbm,
           src_v, dst_v, e_buf, el_v, er_v, ms_v, sp_v,
           part_sh, fin_sh, sem):
    cid = lax.axis_index("c")
    sid = lax.axis_index("s")
    base_e = sid * EP
    pltpu.sync_copy(src_hbm.at[pl.ds(base_e, EP)], src_v)
    pltpu.sync_copy(dst_hbm.at[pl.ds(base_e, EP)], dst_v)

    for h in range(HPS):
      head = cid * HPS + h

      @pl.when(head < H)
      def _head():
        pltpu.sync_copy(eler_hbm.at[pl.ds(head * NP, NP)], el_v)
        pltpu.sync_copy(eler_hbm.at[pl.ds((H + head) * NP, NP)], er_v)

        @pl.loop(0, NV)
        def _initm(j):
          ms_v[pl.ds(j * 16, 16)] = jnp.full((16,), -1e30, jnp.float32)

        @pl.loop(0, VE)
        def _scan1(v):
          sv = src_v[pl.ds(v * 16, 16)]
          dv = dst_v[pl.ds(v * 16, 16)]
          e = plsc.load_gather(el_v, [sv]) + plsc.load_gather(er_v, [dv])
          e = jnp.where(e >= 0.0, e, e * NEG)
          e_buf[pl.ds(v * 16, 16)] = e
          cur = plsc.load_gather(ms_v, [dv])
          plsc.store_scatter(ms_v, [dv], jnp.maximum(cur, e))

        pltpu.sync_copy(ms_v, part_sh.at[sid])
        plsc.subcore_barrier()
        for t in range(NTL):
          pltpu.sync_copy(part_sh.at[t, pl.ds(sid * TS, TS)],
                          er_v.at[pl.ds(t * TS, TS)])

        @pl.loop(0, TS // 16)
        def _redm(j):
          acc = er_v[pl.ds(j * 16, 16)]
          for t in range(1, NTL):
            acc = jnp.maximum(acc, er_v[pl.ds(t * TS + j * 16, 16)])
          ms_v[pl.ds(j * 16, 16)] = acc

        pltpu.sync_copy(ms_v.at[pl.ds(0, TS)], fin_sh.at[pl.ds(sid * TS, TS)])
        plsc.subcore_barrier()
        pltpu.sync_copy(fin_sh, ms_v)
        plsc.subcore_barrier()

        @pl.loop(0, NV)
        def _inits(j):
          sp_v[pl.ds(j * 16, 16)] = jnp.zeros((16,), jnp.float32)

        @pl.loop(0, VE)
        def _scan2(v):
          dv = dst_v[pl.ds(v * 16, 16)]
          ee = jnp.exp(e_buf[pl.ds(v * 16, 16)] - plsc.load_gather(ms_v, [dv]))
          e_buf[pl.ds(v * 16, 16)] = ee
          plsc.addupdate_scatter(sp_v, [dv], ee)

        pltpu.sync_copy(sp_v, part_sh.at[sid])
        plsc.subcore_barrier()
        for t in range(NTL):
          pltpu.sync_copy(part_sh.at[t, pl.ds(sid * TS, TS)],
                          er_v.at[pl.ds(t * TS, TS)])

        @pl.loop(0, TS // 16)
        def _reds(j):
          acc = er_v[pl.ds(j * 16, 16)]
          for t in range(1, NTL):
            acc = acc + er_v[pl.ds(t * TS + j * 16, 16)]
          sp_v[pl.ds(j * 16, 16)] = acc

        pltpu.sync_copy(sp_v.at[pl.ds(0, TS)], fin_sh.at[pl.ds(sid * TS, TS)])
        plsc.subcore_barrier()
        pltpu.sync_copy(fin_sh, sp_v)
        plsc.subcore_barrier()

        for half in range(2):
          @pl.loop(0, VE // 2)
          def _scan3(v):
            off = half * (EP // 2) + v * 16
            dv = dst_v[pl.ds(off, 16)]
            s = plsc.load_gather(sp_v, [dv])
            el_v[pl.ds(v * 16, 16)] = e_buf[pl.ds(off, 16)] / (s + 1e-9)
          pltpu.sync_copy(
              el_v.at[pl.ds(0, EP // 2)],
              out_hbm.at[pl.ds(head * E + base_e + half * (EP // 2), EP // 2)])

  kern = pl.kernel(
      body,
      out_type=jax.ShapeDtypeStruct((H * E,), jnp.float32),
      mesh=_MESH,
      compiler_params=_SC_PARAMS,
      scratch_types=[
          pltpu.VMEM((EP,), jnp.int32),
          pltpu.VMEM((EP,), jnp.int32),
          pltpu.VMEM((EP,), jnp.float32),
          pltpu.VMEM((NP,), jnp.float32),
          pltpu.VMEM((NP,), jnp.float32),
          pltpu.VMEM((NP,), jnp.float32),
          pltpu.VMEM((NP,), jnp.float32),
          pltpu.VMEM_SHARED((NTL, NP), jnp.float32),
          pltpu.VMEM_SHARED((NP,), jnp.float32),
          pltpu.SemaphoreType.DMA,
      ],
  )
  return kern


def _make_agg(F, H, CS, NCH, has_init):
  EP = E // (NSC * NTL)
  VE = EP // LANES
  RB = 32
  D = F // H
  RPT = CS // NTL

  def body(*refs):
    if has_init:
      (src_hbm, dst_hbm, a_hbm, feat_hbm, init_hbm, out_hbm,
       src_v, dst_v, qsrc, qldst, qeid, rows_v, a0q_v, gi_v, si_v, ai_v,
       zero_v, acc_sh, sem) = refs
    else:
      (src_hbm, dst_hbm, a_hbm, feat_hbm, out_hbm,
       src_v, dst_v, qsrc, qldst, qeid, rows_v, a0q_v, gi_v, si_v, ai_v,
       zero_v, acc_sh, sem) = refs
    cid = lax.axis_index("c")
    sid = lax.axis_index("s")
    wid = cid * NTL + sid
    base_e = wid * EP
    pltpu.sync_copy(src_hbm.at[pl.ds(base_e, EP)], src_v)
    pltpu.sync_copy(dst_hbm.at[pl.ds(base_e, EP)], dst_v)

    for r in range(16):
      @pl.loop(0, F // 16)
      def _z(i):
        zero_v[r, pl.ds(i * 16, 16)] = jnp.zeros((16,), jnp.float32)

    iota = lax.iota(jnp.int32, 16)

    for ch in range(NCH):
      base = ch * CS

      @pl.when((ch % NSC) == cid)
      def _chunk():
        if has_init:
          for k in range(RPT // 16):
            pltpu.sync_copy(
                init_hbm.at[pl.ds(base + sid * RPT + k * 16, 16)],
                acc_sh.at[pl.ds(sid * RPT + k * 16, 16)])
        else:
          for k in range(RPT // 16):
            pltpu.sync_copy(zero_v, acc_sh.at[pl.ds(sid * RPT + k * 16, 16)])
        plsc.subcore_barrier()

        def _scan(v, qcnt):
          dv = dst_v[pl.ds(v * 16, 16)]
          msk = (dv >= base) & (dv < base + CS)
          pos = qcnt + plsc.cumsum(jnp.where(msk, 1, 0).astype(jnp.int32)) - 1
          plsc.store_scatter(qsrc, [pos], src_v[pl.ds(v * 16, 16)], mask=msk)
          plsc.store_scatter(qldst, [pos], dv - base, mask=msk)
          plsc.store_scatter(qeid, [pos], base_e + v * 16 + iota, mask=msk)
          return qcnt + plsc.all_reduce_population_count(msk)

        qcnt = lax.fori_loop(0, VE, _scan, jnp.zeros((16,), jnp.int32))
        qn = qcnt[0]
        for pv in range(RB // 16):
          plsc.store_scatter(qsrc, [qn + pv * 16 + iota],
                             jnp.zeros((16,), jnp.int32))
          plsc.store_scatter(qldst, [qn + pv * 16 + iota],
                             jnp.full((16,), CS, jnp.int32))
          plsc.store_scatter(qeid, [qn + pv * 16 + iota],
                             jnp.zeros((16,), jnp.int32))

        nb = (qn + RB - 1) // RB

        def _batch(b, _):
          q0 = b * RB
          for pv in range(RB // 16):
            gi_v[pl.ds(pv * 16, 16)] = qsrc[pl.ds(q0 + pv * 16, 16)]
            si_v[pl.ds(pv * 16, 16)] = qldst[pl.ds(q0 + pv * 16, 16)]
            ev = qeid[pl.ds(q0 + pv * 16, 16)]
            for h in range(H):
              ai_v[h, pl.ds(pv * 16, 16)] = ev + h * E
          cp = pltpu.async_copy(feat_hbm.at[gi_v], rows_v, sem)
          acs = [pltpu.async_copy(a_hbm.at[ai_v.at[h]],
                                  a0q_v.at[pl.ds(h * RB, RB)], sem)
                 for h in range(H)]
          cp.wait()
          for ac in acs:
            ac.wait()

          def _srow(r, _):
            for h in range(H):
              scale = plsc.load_gather(
                  a0q_v, [jnp.full((16,), h * RB + r, jnp.int32)])
              for j in range(D // 16):
                col = h * D + j * 16
                rows_v[r, pl.ds(col, 16)] = rows_v[r, pl.ds(col, 16)] * scale
            return 0

          lax.fori_loop(0, RB, _srow, 0)
          pltpu.sync_copy(rows_v, acc_sh.at[si_v], add=True)
          return 0

        lax.fori_loop(0, nb, _batch, 0)
        plsc.subcore_barrier()

        for k in range(RPT // 16):
          pltpu.sync_copy(acc_sh.at[pl.ds(sid * RPT + k * 16, 16)],
                          out_hbm.at[pl.ds(base + sid * RPT + k * 16, 16)])
        plsc.subcore_barrier()

  scratch = [
      pltpu.VMEM((EP,), jnp.int32),
      pltpu.VMEM((EP,), jnp.int32),
      pltpu.VMEM((EP + RB,), jnp.int32),
      pltpu.VMEM((EP + RB,), jnp.int32),
      pltpu.VMEM((EP + RB,), jnp.int32),
      pltpu.VMEM((RB, F), jnp.float32),
      pltpu.VMEM((H * RB,), jnp.float32),
      pltpu.VMEM((RB,), jnp.int32),
      pltpu.VMEM((RB,), jnp.int32),
      pltpu.VMEM((H, RB), jnp.int32),
      pltpu.VMEM((16, F), jnp.float32),
      pltpu.VMEM_SHARED((CS + 8, F), jnp.float32),
      pltpu.SemaphoreType.DMA,
  ]
  kern = pl.kernel(
      body,
      out_type=jax.ShapeDtypeStruct((NP, F), jnp.float32),
      mesh=_MESH,
      compiler_params=_SC_PARAMS,
      scratch_types=scratch,
  )
  return kern


_ATTN0 = _make_attn(H0)
_ATTN1 = _make_attn(H1)
_AGG0 = _make_agg(H0 * HID, H0, 1024, NP // 1024, False)
_AGG1 = _make_agg(CLS, H1, 2560, NP // 2560, True)


def kernel(inputs, edge_index, W0, al0, ar0, W1, al1, ar1, resW1):
  src = edge_index[0]
  dst = edge_index[1]
  x_pad = jnp.pad(inputs, ((0, NP - N), (0, 0)))

  feat0, eler0 = _tc1(x_pad, W0, al0, ar0)
  elerT0 = eler0.T.reshape(-1)
  a0_flat = _ATTN0(src, dst, elerT0)
  h_pre = _AGG0(src, dst, a0_flat, feat0)

  Wcat = jnp.concatenate([W1, resW1], axis=1)
  feat1, res1, eler1 = _tc2(h_pre, Wcat, al1, ar1)
  elerT1 = eler1.T.reshape(-1)
  a1_flat = _ATTN1(src, dst, elerT1)
  out1 = _AGG1(src, dst, a1_flat, feat1, res1)

  logits = out1[:N]
  a0 = a0_flat.reshape(H0, E).T
  a1 = a1_flat.reshape(E, H1)
  return logits, a0, a1


# speedup vs baseline: 13.6202x; 13.6202x over previous
import functools

import jax
import jax.numpy as jnp
from jax import lax
from jax.experimental import pallas as pl
from jax.experimental.pallas import tpu as pltpu
from jax.experimental.pallas import tpu_sc as plsc

N = 10000
E = 320000
IN_DIM = 128
HID = 128
H0 = 8
H1 = 1
CLS = 64
NEG = 0.2

NP = 10240
NSC = 2
NTL = 16
LANES = 16

_MESH = plsc.VectorSubcoreMesh(core_axis_name="c", subcore_axis_name="s")
_SC_PARAMS = pltpu.CompilerParams(needs_layout_passes=False)


def _tc1_body(x_ref, w_ref, al_ref, ar_ref, feat_ref, eler_ref):
  x = x_ref[...]
  p = jnp.dot(x, w_ref[...], preferred_element_type=jnp.float32)
  feat_ref[...] = p
  fr = p.reshape(p.shape[0], H0, HID)
  el = jnp.sum(fr * al_ref[...][None], axis=-1)
  er = jnp.sum(fr * ar_ref[...][None], axis=-1)
  eler_ref[...] = jnp.concatenate([el, er], axis=1)


def _tc1(x_pad, W0, al0, ar0):
  blk = 512
  grid = (NP // blk,)
  return pl.pallas_call(
      _tc1_body,
      grid=grid,
      in_specs=[
          pl.BlockSpec((blk, IN_DIM), lambda i: (i, 0)),
          pl.BlockSpec((IN_DIM, H0 * HID), lambda i: (0, 0)),
          pl.BlockSpec((H0, HID), lambda i: (0, 0)),
          pl.BlockSpec((H0, HID), lambda i: (0, 0)),
      ],
      out_specs=[
          pl.BlockSpec((blk, H0 * HID), lambda i: (i, 0)),
          pl.BlockSpec((blk, 2 * H0), lambda i: (i, 0)),
      ],
      out_shape=[
          jax.ShapeDtypeStruct((NP, H0 * HID), jnp.float32),
          jax.ShapeDtypeStruct((NP, 2 * H0), jnp.float32),
      ],
  )(x_pad, W0, al0, ar0)


def _tc2_body(h_ref, w_ref, al_ref, ar_ref, feat_ref, res_ref, eler_ref):
  x = jnp.maximum(h_ref[...], 0.0)
  p = jnp.dot(x, w_ref[...], preferred_element_type=jnp.float32)
  f = p[:, :CLS]
  z = jnp.zeros_like(f)
  feat_ref[...] = jnp.concatenate([f, z], axis=1)
  res_ref[...] = jnp.concatenate([p[:, CLS:], z], axis=1)
  el = jnp.sum(f * al_ref[...], axis=-1, keepdims=True)
  er = jnp.sum(f * ar_ref[...], axis=-1, keepdims=True)
  eler_ref[...] = jnp.concatenate([el, er], axis=1)


def _tc2(h_pad, Wcat, al1, ar1):
  blk = 512
  grid = (NP // blk,)
  return pl.pallas_call(
      _tc2_body,
      grid=grid,
      in_specs=[
          pl.BlockSpec((blk, H0 * HID), lambda i: (i, 0)),
          pl.BlockSpec((H0 * HID, 2 * CLS), lambda i: (0, 0)),
          pl.BlockSpec((1, CLS), lambda i: (0, 0)),
          pl.BlockSpec((1, CLS), lambda i: (0, 0)),
      ],
      out_specs=[
          pl.BlockSpec((blk, 2 * CLS), lambda i: (i, 0)),
          pl.BlockSpec((blk, 2 * CLS), lambda i: (i, 0)),
          pl.BlockSpec((blk, 2), lambda i: (i, 0)),
      ],
      out_shape=[
          jax.ShapeDtypeStruct((NP, 2 * CLS), jnp.float32),
          jax.ShapeDtypeStruct((NP, 2 * CLS), jnp.float32),
          jax.ShapeDtypeStruct((NP, 2), jnp.float32),
      ],
  )(h_pad, Wcat, al1, ar1)


def _make_attn(H):
  HPS = max(H // NSC, 1)
  EP = E // NTL
  VE = EP // LANES
  TS = NP // NTL
  NV = NP // LANES

  def body(src_hbm, dst_hbm, eler_hbm, out_hbm,
           src_v, dst_v, e_buf, el_v, er_v, ms_v, sp_v,
           part_sh, fin_sh, sem):
    cid = lax.axis_index("c")
    sid = lax.axis_index("s")
    base_e = sid * EP
    pltpu.sync_copy(src_hbm.at[pl.ds(base_e, EP)], src_v)
    pltpu.sync_copy(dst_hbm.at[pl.ds(base_e, EP)], dst_v)

    for h in range(HPS):
      head = cid * HPS + h

      @pl.when(head < H)
      def _head():
        pltpu.sync_copy(eler_hbm.at[pl.ds(head * NP, NP)], el_v)
        pltpu.sync_copy(eler_hbm.at[pl.ds((H + head) * NP, NP)], er_v)

        @pl.loop(0, NV)
        def _initm(j):
          ms_v[pl.ds(j * 16, 16)] = jnp.full((16,), -1e30, jnp.float32)

        @pl.loop(0, VE)
        def _scan1(v):
          sv = src_v[pl.ds(v * 16, 16)]
          dv = dst_v[pl.ds(v * 16, 16)]
          e = plsc.load_gather(el_v, [sv]) + plsc.load_gather(er_v, [dv])
          e = jnp.where(e >= 0.0, e, e * NEG)
          e_buf[pl.ds(v * 16, 16)] = e
          cur = plsc.load_gather(ms_v, [dv])
          plsc.store_scatter(ms_v, [dv], jnp.maximum(cur, e))

        pltpu.sync_copy(ms_v, part_sh.at[sid])
        plsc.subcore_barrier()
        for t in range(NTL):
          pltpu.sync_copy(part_sh.at[t, pl.ds(sid * TS, TS)],
                          er_v.at[pl.ds(t * TS, TS)])

        @pl.loop(0, TS // 16)
        def _redm(j):
          acc = er_v[pl.ds(j * 16, 16)]
          for t in range(1, NTL):
            acc = jnp.maximum(acc, er_v[pl.ds(t * TS + j * 16, 16)])
          ms_v[pl.ds(j * 16, 16)] = acc

        pltpu.sync_copy(ms_v.at[pl.ds(0, TS)], fin_sh.at[pl.ds(sid * TS, TS)])
        plsc.subcore_barrier()
        pltpu.sync_copy(fin_sh, ms_v)
        plsc.subcore_barrier()

        @pl.loop(0, NV)
        def _inits(j):
          sp_v[pl.ds(j * 16, 16)] = jnp.zeros((16,), jnp.float32)

        @pl.loop(0, VE)
        def _scan2(v):
          dv = dst_v[pl.ds(v * 16, 16)]
          ee = jnp.exp(e_buf[pl.ds(v * 16, 16)] - plsc.load_gather(ms_v, [dv]))
          e_buf[pl.ds(v * 16, 16)] = ee
          plsc.addupdate_scatter(sp_v, [dv], ee)

        pltpu.sync_copy(sp_v, part_sh.at[sid])
        plsc.subcore_barrier()
        for t in range(NTL):
          pltpu.sync_copy(part_sh.at[t, pl.ds(sid * TS, TS)],
                          er_v.at[pl.ds(t * TS, TS)])

        @pl.loop(0, TS // 16)
        def _reds(j):
          acc = er_v[pl.ds(j * 16, 16)]
          for t in range(1, NTL):
            acc = acc + er_v[pl.ds(t * TS + j * 16, 16)]
          sp_v[pl.ds(j * 16, 16)] = acc

        pltpu.sync_copy(sp_v.at[pl.ds(0, TS)], fin_sh.at[pl.ds(sid * TS, TS)])
        plsc.subcore_barrier()
        pltpu.sync_copy(fin_sh, sp_v)
        plsc.subcore_barrier()

        for half in range(2):
          @pl.loop(0, VE // 2)
          def _scan3(v):
            off = half * (EP // 2) + v * 16
            dv = dst_v[pl.ds(off, 16)]
            s = plsc.load_gather(sp_v, [dv])
            el_v[pl.ds(v * 16, 16)] = e_buf[pl.ds(off, 16)] / (s + 1e-9)
          pltpu.sync_copy(
              el_v.at[pl.ds(0, EP // 2)],
              out_hbm.at[pl.ds(head * E + base_e + half * (EP // 2), EP // 2)])

  kern = pl.kernel(
      body,
      out_type=jax.ShapeDtypeStruct((H * E,), jnp.float32),
      mesh=_MESH,
      compiler_params=_SC_PARAMS,
      scratch_types=[
          pltpu.VMEM((EP,), jnp.int32),
          pltpu.VMEM((EP,), jnp.int32),
          pltpu.VMEM((EP,), jnp.float32),
          pltpu.VMEM((NP,), jnp.float32),
          pltpu.VMEM((NP,), jnp.float32),
          pltpu.VMEM((NP,), jnp.float32),
          pltpu.VMEM((NP,), jnp.float32),
          pltpu.VMEM_SHARED((NTL, NP), jnp.float32),
          pltpu.VMEM_SHARED((NP,), jnp.float32),
          pltpu.SemaphoreType.DMA,
      ],
  )
  return kern


def _make_agg(GR, H, CS, NCH, has_init):
  EP = E // NTL
  VE = EP // LANES
  RB = 16
  DJ = (128 // GR) // 16 if H == 1 else 128 // 16
  RPT = CS // NTL
  FR = RPT * GR

  def body(*refs):
    if has_init:
      (src_hbm, dst_hbm, a_hbm, feat_hbm, init_hbm, out_hbm,
       src_v, dst_v, qpk, rows_v, a0q_v, gi_v, si_v, ai_v,
       zero_v, acc_sh, sem, sem2) = refs
    else:
      (src_hbm, dst_hbm, a_hbm, feat_hbm, out_hbm,
       src_v, dst_v, qpk, rows_v, a0q_v, gi_v, si_v, ai_v,
       zero_v, acc_sh, sem, sem2) = refs
    cid = lax.axis_index("c")
    sid = lax.axis_index("s")
    base_e = sid * EP
    pltpu.sync_copy(src_hbm.at[pl.ds(base_e, EP)], src_v)
    pltpu.sync_copy(dst_hbm.at[pl.ds(base_e, EP)], dst_v)

    for r in range(8):
      @pl.loop(0, 8)
      def _z(i):
        zero_v[r, pl.ds(i * 16, 16)] = jnp.zeros((16,), jnp.float32)

    iota = lax.iota(jnp.int32, 16)

    @pl.loop(0, NCH)
    def _chunk_loop(ch):
      base = ch * CS

      @pl.when((ch % NSC) == cid)
      def _chunk():
        if has_init:
          for k in range(FR // 8):
            pltpu.sync_copy(
                init_hbm.at[pl.ds((base * GR) + sid * FR + k * 8, 8)],
                acc_sh.at[pl.ds(sid * FR + k * 8, 8)])
        else:
          for k in range(FR // 8):
            pltpu.sync_copy(zero_v, acc_sh.at[pl.ds(sid * FR + k * 8, 8)])
        plsc.subcore_barrier()

        def _scan(v, qcnt):
          dv = dst_v[pl.ds(v * 16, 16)]
          msk = (dv >= base) & (dv < base + CS)
          pos = qcnt + plsc.cumsum(jnp.where(msk, 1, 0).astype(jnp.int32)) - 1
          qval = ((v * 16 + iota) << 12) | (dv - base)
          plsc.store_scatter(qpk, [pos], qval, mask=msk)
          return qcnt + plsc.all_reduce_population_count(msk)

        qcnt = lax.fori_loop(0, VE, _scan, jnp.zeros((16,), jnp.int32))
        qn = qcnt[0]
        plsc.store_scatter(qpk, [qn + iota], jnp.full((16,), CS, jnp.int32))

        nb = (qn + RB - 1) // RB

        def _batch(b, _):
          q0 = b * RB
          if True:
            qv = qpk[pl.ds(q0, 16)]
            rel = qv >> 12
            ldst = qv & 4095
            sv = plsc.load_gather(src_v, [rel])
            epos = iota * GR
            for k in range(GR):
              plsc.store_scatter(gi_v, [epos + k], sv * GR + k)
              plsc.store_scatter(si_v, [epos + k], ldst * GR + k)
            for h in range(H):
              ai_v[h, pl.ds(0, 16)] = rel + (base_e + h * E)
          cp = pltpu.async_copy(feat_hbm.at[gi_v], rows_v, sem)
          acs = [pltpu.async_copy(a_hbm.at[ai_v.at[h]],
                                  a0q_v.at[pl.ds(h * RB, RB)], sem2)
                 for h in range(H)]
          cp.wait()
          for ac in acs:
            ac.wait()

          def _srow(r, _):
            for h in range(H):
              scale = plsc.load_gather(
                  a0q_v, [jnp.full((16,), h * RB + r, jnp.int32)])
              for j in range(DJ):
                rows_v[r * GR + h, pl.ds(j * 16, 16)] = (
                    rows_v[r * GR + h, pl.ds(j * 16, 16)] * scale)
            return 0

          lax.fori_loop(0, RB, _srow, 0)
          pltpu.sync_copy(rows_v, acc_sh.at[si_v], add=True)
          return 0

        lax.fori_loop(0, nb, _batch, 0)
        plsc.subcore_barrier()

        for k in range(FR // 16):
          pltpu.sync_copy(acc_sh.at[pl.ds(sid * FR + k * 16, 16)],
                          out_hbm.at[pl.ds(base * GR + sid * FR + k * 16, 16)])
        plsc.subcore_barrier()

  scratch = [
      pltpu.VMEM((EP,), jnp.int32),
      pltpu.VMEM((EP,), jnp.int32),
      pltpu.VMEM((EP + RB,), jnp.int32),
      pltpu.VMEM((RB * GR, 128), jnp.float32),
      pltpu.VMEM((H * RB,), jnp.float32),
      pltpu.VMEM((RB * GR,), jnp.int32),
      pltpu.VMEM((RB * GR,), jnp.int32),
      pltpu.VMEM((H, RB), jnp.int32),
      pltpu.VMEM((8, 128), jnp.float32),
      pltpu.VMEM_SHARED((CS * GR + 64, 128), jnp.float32),
      pltpu.SemaphoreType.DMA,
      pltpu.SemaphoreType.DMA,
  ]
  kern = pl.kernel(
      body,
      out_type=jax.ShapeDtypeStruct((NP * GR, 128), jnp.float32),
      mesh=_MESH,
      compiler_params=_SC_PARAMS,
      scratch_types=scratch,
  )
  return kern


_ATTN0 = _make_attn(H0)
_ATTN1 = _make_attn(H1)
_AGG0 = _make_agg(H0, H0, 640, NP // 640, False)
_AGG1 = _make_agg(1, H1, 2560, NP // 2560, True)


def kernel(inputs, edge_index, W0, al0, ar0, W1, al1, ar1, resW1):
  src = edge_index[0]
  dst = edge_index[1]
  x_pad = jnp.pad(inputs, ((0, NP - N), (0, 0)))

  feat0, eler0 = _tc1(x_pad, W0, al0, ar0)
  elerT0 = eler0.T.reshape(-1)
  a0_flat = _ATTN0(src, dst, elerT0)
  h_prer = _AGG0(src, dst, a0_flat, feat0.reshape(NP * H0, HID))
  h_pre = h_prer.reshape(NP, H0 * HID)

  Wcat = jnp.concatenate([W1, resW1], axis=1)
  feat1, res1, eler1 = _tc2(h_pre, Wcat, al1, ar1)
  elerT1 = eler1.T.reshape(-1)
  a1_flat = _ATTN1(src, dst, elerT1)
  out1 = _AGG1(src, dst, a1_flat, feat1, res1)

  logits = out1[:N, :CLS]
  a0 = a0_flat.reshape(H0, E).T
  a1 = a1_flat.reshape(E, H1)
  return logits, a0, a1


# speedup vs baseline: 13.6717x; 1.0038x over previous
import functools

import jax
import jax.numpy as jnp
from jax import lax
from jax.experimental import pallas as pl
from jax.experimental.pallas import tpu as pltpu
from jax.experimental.pallas import tpu_sc as plsc

N = 10000
E = 320000
IN_DIM = 128
HID = 128
H0 = 8
H1 = 1
CLS = 64
NEG = 0.2

NP = 10240
NSC = 2
NTL = 16
LANES = 16

_MESH = plsc.VectorSubcoreMesh(core_axis_name="c", subcore_axis_name="s")
_SC_PARAMS = pltpu.CompilerParams(needs_layout_passes=False)


def _tc1_body(x_ref, w_ref, al_ref, ar_ref, feat_ref, eler_ref):
  x = x_ref[...]
  p = jnp.dot(x, w_ref[...], preferred_element_type=jnp.float32)
  feat_ref[...] = p
  fr = p.reshape(p.shape[0], H0, HID)
  el = jnp.sum(fr * al_ref[...][None], axis=-1)
  er = jnp.sum(fr * ar_ref[...][None], axis=-1)
  eler_ref[...] = jnp.concatenate([el, er], axis=1)


def _tc1(x_pad, W0, al0, ar0):
  blk = 512
  grid = (NP // blk,)
  return pl.pallas_call(
      _tc1_body,
      grid=grid,
      in_specs=[
          pl.BlockSpec((blk, IN_DIM), lambda i: (i, 0)),
          pl.BlockSpec((IN_DIM, H0 * HID), lambda i: (0, 0)),
          pl.BlockSpec((H0, HID), lambda i: (0, 0)),
          pl.BlockSpec((H0, HID), lambda i: (0, 0)),
      ],
      out_specs=[
          pl.BlockSpec((blk, H0 * HID), lambda i: (i, 0)),
          pl.BlockSpec((blk, 2 * H0), lambda i: (i, 0)),
      ],
      out_shape=[
          jax.ShapeDtypeStruct((NP, H0 * HID), jnp.float32),
          jax.ShapeDtypeStruct((NP, 2 * H0), jnp.float32),
      ],
  )(x_pad, W0, al0, ar0)


def _tc2_body(h_ref, w_ref, al_ref, ar_ref, feat_ref, res_ref, eler_ref):
  x = jnp.maximum(h_ref[...], 0.0)
  p = jnp.dot(x, w_ref[...], preferred_element_type=jnp.float32)
  f = p[:, :CLS]
  z = jnp.zeros_like(f)
  feat_ref[...] = jnp.concatenate([f, z], axis=1)
  res_ref[...] = jnp.concatenate([p[:, CLS:], z], axis=1)
  el = jnp.sum(f * al_ref[...], axis=-1, keepdims=True)
  er = jnp.sum(f * ar_ref[...], axis=-1, keepdims=True)
  eler_ref[...] = jnp.concatenate([el, er], axis=1)


def _tc2(h_pad, Wcat, al1, ar1):
  blk = 512
  grid = (NP // blk,)
  return pl.pallas_call(
      _tc2_body,
      grid=grid,
      in_specs=[
          pl.BlockSpec((blk, H0 * HID), lambda i: (i, 0)),
          pl.BlockSpec((H0 * HID, 2 * CLS), lambda i: (0, 0)),
          pl.BlockSpec((1, CLS), lambda i: (0, 0)),
          pl.BlockSpec((1, CLS), lambda i: (0, 0)),
      ],
      out_specs=[
          pl.BlockSpec((blk, 2 * CLS), lambda i: (i, 0)),
          pl.BlockSpec((blk, 2 * CLS), lambda i: (i, 0)),
          pl.BlockSpec((blk, 2), lambda i: (i, 0)),
      ],
      out_shape=[
          jax.ShapeDtypeStruct((NP, 2 * CLS), jnp.float32),
          jax.ShapeDtypeStruct((NP, 2 * CLS), jnp.float32),
          jax.ShapeDtypeStruct((NP, 2), jnp.float32),
      ],
  )(h_pad, Wcat, al1, ar1)


def _make_attn(H):
  HPS = max(H // NSC, 1)
  EP = E // NTL
  VE = EP // LANES
  TS = NP // NTL
  NV = NP // LANES

  def body(src_hbm, dst_hbm, eler_hbm, out_hbm,
           src_v, dst_v, e_buf, el_v, er_v, ms_v, sp_v,
           part_sh, fin_sh, sem):
    cid = lax.axis_index("c")
    sid = lax.axis_index("s")
    base_e = sid * EP
    pltpu.sync_copy(src_hbm.at[pl.ds(base_e, EP)], src_v)
    pltpu.sync_copy(dst_hbm.at[pl.ds(base_e, EP)], dst_v)

    for h in range(HPS):
      head = cid * HPS + h

      @pl.when(head < H)
      def _head():
        pltpu.sync_copy(eler_hbm.at[pl.ds(head * NP, NP)], el_v)
        pltpu.sync_copy(eler_hbm.at[pl.ds((H + head) * NP, NP)], er_v)

        @pl.loop(0, NV)
        def _initm(j):
          ms_v[pl.ds(j * 16, 16)] = jnp.full((16,), -1e30, jnp.float32)

        @pl.loop(0, VE)
        def _scan1(v):
          sv = src_v[pl.ds(v * 16, 16)]
          dv = dst_v[pl.ds(v * 16, 16)]
          e = plsc.load_gather(el_v, [sv]) + plsc.load_gather(er_v, [dv])
          e = jnp.where(e >= 0.0, e, e * NEG)
          e_buf[pl.ds(v * 16, 16)] = e
          cur = plsc.load_gather(ms_v, [dv])
          plsc.store_scatter(ms_v, [dv], jnp.maximum(cur, e))

        pltpu.sync_copy(ms_v, part_sh.at[sid])
        plsc.subcore_barrier()
        for t in range(NTL):
          pltpu.sync_copy(part_sh.at[t, pl.ds(sid * TS, TS)],
                          er_v.at[pl.ds(t * TS, TS)])

        @pl.loop(0, TS // 16)
        def _redm(j):
          acc = er_v[pl.ds(j * 16, 16)]
          for t in range(1, NTL):
            acc = jnp.maximum(acc, er_v[pl.ds(t * TS + j * 16, 16)])
          ms_v[pl.ds(j * 16, 16)] = acc

        pltpu.sync_copy(ms_v.at[pl.ds(0, TS)], fin_sh.at[pl.ds(sid * TS, TS)])
        plsc.subcore_barrier()
        pltpu.sync_copy(fin_sh, ms_v)
        plsc.subcore_barrier()

        @pl.loop(0, NV)
        def _inits(j):
          sp_v[pl.ds(j * 16, 16)] = jnp.zeros((16,), jnp.float32)

        @pl.loop(0, VE)
        def _scan2(v):
          dv = dst_v[pl.ds(v * 16, 16)]
          ee = jnp.exp(e_buf[pl.ds(v * 16, 16)] - plsc.load_gather(ms_v, [dv]))
          e_buf[pl.ds(v * 16, 16)] = ee
          plsc.addupdate_scatter(sp_v, [dv], ee)

        pltpu.sync_copy(sp_v, part_sh.at[sid])
        plsc.subcore_barrier()
        for t in range(NTL):
          pltpu.sync_copy(part_sh.at[t, pl.ds(sid * TS, TS)],
                          er_v.at[pl.ds(t * TS, TS)])

        @pl.loop(0, TS // 16)
        def _reds(j):
          acc = er_v[pl.ds(j * 16, 16)]
          for t in range(1, NTL):
            acc = acc + er_v[pl.ds(t * TS + j * 16, 16)]
          sp_v[pl.ds(j * 16, 16)] = acc

        pltpu.sync_copy(sp_v.at[pl.ds(0, TS)], fin_sh.at[pl.ds(sid * TS, TS)])
        plsc.subcore_barrier()
        pltpu.sync_copy(fin_sh, sp_v)
        plsc.subcore_barrier()

        for half in range(2):
          @pl.loop(0, VE // 2)
          def _scan3(v):
            off = half * (EP // 2) + v * 16
            dv = dst_v[pl.ds(off, 16)]
            s = plsc.load_gather(sp_v, [dv])
            el_v[pl.ds(v * 16, 16)] = e_buf[pl.ds(off, 16)] / (s + 1e-9)
          pltpu.sync_copy(
              el_v.at[pl.ds(0, EP // 2)],
              out_hbm.at[pl.ds(head * E + base_e + half * (EP // 2), EP // 2)])

  kern = pl.kernel(
      body,
      out_type=jax.ShapeDtypeStruct((H * E,), jnp.float32),
      mesh=_MESH,
      compiler_params=_SC_PARAMS,
      scratch_types=[
          pltpu.VMEM((EP,), jnp.int32),
          pltpu.VMEM((EP,), jnp.int32),
          pltpu.VMEM((EP,), jnp.float32),
          pltpu.VMEM((NP,), jnp.float32),
          pltpu.VMEM((NP,), jnp.float32),
          pltpu.VMEM((NP,), jnp.float32),
          pltpu.VMEM((NP,), jnp.float32),
          pltpu.VMEM_SHARED((NTL, NP), jnp.float32),
          pltpu.VMEM_SHARED((NP,), jnp.float32),
          pltpu.SemaphoreType.DMA,
      ],
  )
  return kern


def _make_agg(GR, H, CS, NCH, has_init):
  EP = E // NTL
  VE = EP // LANES
  RB = 16
  DJ = (128 // GR) // 16 if H == 1 else 128 // 16
  RPT = CS // NTL
  FR = RPT * GR

  def body(*refs):
    if has_init:
      (src_hbm, dst_hbm, a_hbm, feat_hbm, init_hbm, out_hbm,
       src_v, dst_v, qpk, rows_v, a0q_v, gi_v, si_v, ai_v,
       zero_v, acc_sh, sem, sem2) = refs
    else:
      (src_hbm, dst_hbm, a_hbm, feat_hbm, out_hbm,
       src_v, dst_v, qpk, rows_v, a0q_v, gi_v, si_v, ai_v,
       zero_v, acc_sh, sem, sem2) = refs
    cid = lax.axis_index("c")
    sid = lax.axis_index("s")
    base_e = sid * EP
    pltpu.sync_copy(src_hbm.at[pl.ds(base_e, EP)], src_v)
    pltpu.sync_copy(dst_hbm.at[pl.ds(base_e, EP)], dst_v)

    for r in range(8):
      @pl.loop(0, 8)
      def _z(i):
        zero_v[r, pl.ds(i * 16, 16)] = jnp.zeros((16,), jnp.float32)

    iota = lax.iota(jnp.int32, 16)

    @pl.loop(0, NCH)
    def _chunk_loop(ch):
      base = ch * CS

      @pl.when((ch % NSC) == cid)
      def _chunk():
        if has_init:
          for k in range(FR // 8):
            pltpu.sync_copy(
                init_hbm.at[pl.ds((base * GR) + sid * FR + k * 8, 8)],
                acc_sh.at[pl.ds(sid * FR + k * 8, 8)])
        else:
          for k in range(FR // 8):
            pltpu.sync_copy(zero_v, acc_sh.at[pl.ds(sid * FR + k * 8, 8)])
        plsc.subcore_barrier()

        def _scan(v, qcnt):
          dv = dst_v[pl.ds(v * 16, 16)]
          msk = (dv >= base) & (dv < base + CS)
          pos = qcnt + plsc.cumsum(jnp.where(msk, 1, 0).astype(jnp.int32)) - 1
          qval = ((v * 16 + iota) << 12) | (dv - base)
          plsc.store_scatter(qpk, [pos], qval, mask=msk)
          return qcnt + plsc.all_reduce_population_count(msk)

        qcnt = lax.fori_loop(0, VE, _scan, jnp.zeros((16,), jnp.int32))
        qn = qcnt[0]
        plsc.store_scatter(qpk, [qn + iota], jnp.full((16,), CS, jnp.int32))

        nb = (qn + RB - 1) // RB

        def _batch(b, _):
          q0 = b * RB
          if True:
            qv = qpk[pl.ds(q0, 16)]
            rel = qv >> 12
            ldst = qv & 4095
            sv = plsc.load_gather(src_v, [rel])
            epos = iota * GR
            for k in range(GR):
              plsc.store_scatter(gi_v, [epos + k], sv * GR + k)
              plsc.store_scatter(si_v, [epos + k], ldst * GR + k)
            for h in range(H):
              ai_v[pl.ds(h * RB, 16)] = rel + (base_e + h * E)
          cp = pltpu.async_copy(feat_hbm.at[gi_v], rows_v, sem)
          ac = pltpu.async_copy(a_hbm.at[ai_v], a0q_v, sem2)
          cp.wait()
          ac.wait()

          def _srow(r, _):
            for h in range(H):
              scale = plsc.load_gather(
                  a0q_v, [jnp.full((16,), h * RB + r, jnp.int32)])
              for j in range(DJ):
                rows_v[r * GR + h, pl.ds(j * 16, 16)] = (
                    rows_v[r * GR + h, pl.ds(j * 16, 16)] * scale)
            return 0

          lax.fori_loop(0, RB, _srow, 0)
          pltpu.sync_copy(rows_v, acc_sh.at[si_v], add=True)
          return 0

        lax.fori_loop(0, nb, _batch, 0)
        plsc.subcore_barrier()

        for k in range(FR // 16):
          pltpu.sync_copy(acc_sh.at[pl.ds(sid * FR + k * 16, 16)],
                          out_hbm.at[pl.ds(base * GR + sid * FR + k * 16, 16)])
        plsc.subcore_barrier()

  scratch = [
      pltpu.VMEM((EP,), jnp.int32),
      pltpu.VMEM((EP,), jnp.int32),
      pltpu.VMEM((EP + RB,), jnp.int32),
      pltpu.VMEM((RB * GR, 128), jnp.float32),
      pltpu.VMEM((H * RB,), jnp.float32),
      pltpu.VMEM((RB * GR,), jnp.int32),
      pltpu.VMEM((RB * GR,), jnp.int32),
      pltpu.VMEM((H * RB,), jnp.int32),
      pltpu.VMEM((8, 128), jnp.float32),
      pltpu.VMEM_SHARED((CS * GR + 64, 128), jnp.float32),
      pltpu.SemaphoreType.DMA,
      pltpu.SemaphoreType.DMA,
  ]
  kern = pl.kernel(
      body,
      out_type=jax.ShapeDtypeStruct((NP * GR, 128), jnp.float32),
      mesh=_MESH,
      compiler_params=_SC_PARAMS,
      scratch_types=scratch,
  )
  return kern


_ATTN0 = _make_attn(H0)
_ATTN1 = _make_attn(H1)
_AGG0 = _make_agg(H0, H0, 640, NP // 640, False)
_AGG1 = _make_agg(1, H1, 2560, NP // 2560, True)


def kernel(inputs, edge_index, W0, al0, ar0, W1, al1, ar1, resW1):
  src = edge_index[0]
  dst = edge_index[1]
  x_pad = jnp.pad(inputs, ((0, NP - N), (0, 0)))

  feat0, eler0 = _tc1(x_pad, W0, al0, ar0)
  elerT0 = eler0.T.reshape(-1)
  a0_flat = _ATTN0(src, dst, elerT0)
  h_prer = _AGG0(src, dst, a0_flat, feat0.reshape(NP * H0, HID))
  h_pre = h_prer.reshape(NP, H0 * HID)

  Wcat = jnp.concatenate([W1, resW1], axis=1)
  feat1, res1, eler1 = _tc2(h_pre, Wcat, al1, ar1)
  elerT1 = eler1.T.reshape(-1)
  a1_flat = _ATTN1(src, dst, elerT1)
  out1 = _AGG1(src, dst, a1_flat, feat1, res1)

  logits = out1[:N, :CLS]
  a0 = a0_flat.reshape(H0, E).T
  a1 = a1_flat.reshape(E, H1)
  return logits, a0, a1


# speedup vs baseline: 14.4519x; 1.0571x over previous
import functools

import jax
import jax.numpy as jnp
from jax import lax
from jax.experimental import pallas as pl
from jax.experimental.pallas import tpu as pltpu
from jax.experimental.pallas import tpu_sc as plsc

N = 10000
E = 320000
IN_DIM = 128
HID = 128
H0 = 8
H1 = 1
CLS = 64
NEG = 0.2

NP = 10240
NSC = 2
NTL = 16
LANES = 16

_MESH = plsc.VectorSubcoreMesh(core_axis_name="c", subcore_axis_name="s")
_SC_PARAMS = pltpu.CompilerParams(needs_layout_passes=False)


def _tc1_body(x_ref, w_ref, al_ref, ar_ref, feat_ref, eler_ref):
  x = x_ref[...]
  p = jnp.dot(x, w_ref[...], preferred_element_type=jnp.float32)
  feat_ref[...] = p
  fr = p.reshape(p.shape[0], H0, HID)
  el = jnp.sum(fr * al_ref[...][None], axis=-1)
  er = jnp.sum(fr * ar_ref[...][None], axis=-1)
  eler_ref[...] = jnp.concatenate([el, er], axis=1)


def _tc1(x_pad, W0, al0, ar0):
  blk = 512
  grid = (NP // blk,)
  return pl.pallas_call(
      _tc1_body,
      grid=grid,
      in_specs=[
          pl.BlockSpec((blk, IN_DIM), lambda i: (i, 0)),
          pl.BlockSpec((IN_DIM, H0 * HID), lambda i: (0, 0)),
          pl.BlockSpec((H0, HID), lambda i: (0, 0)),
          pl.BlockSpec((H0, HID), lambda i: (0, 0)),
      ],
      out_specs=[
          pl.BlockSpec((blk, H0 * HID), lambda i: (i, 0)),
          pl.BlockSpec((blk, 2 * H0), lambda i: (i, 0)),
      ],
      out_shape=[
          jax.ShapeDtypeStruct((NP, H0 * HID), jnp.float32),
          jax.ShapeDtypeStruct((NP, 2 * H0), jnp.float32),
      ],
  )(x_pad, W0, al0, ar0)


def _tc2_body(h_ref, w_ref, al_ref, ar_ref, feat_ref, res_ref, eler_ref):
  x = jnp.maximum(h_ref[...], 0.0)
  p = jnp.dot(x, w_ref[...], preferred_element_type=jnp.float32)
  f = p[:, :CLS]
  z = jnp.zeros_like(f)
  feat_ref[...] = jnp.concatenate([f, z], axis=1)
  res_ref[...] = jnp.concatenate([p[:, CLS:], z], axis=1)
  el = jnp.sum(f * al_ref[...], axis=-1, keepdims=True)
  er = jnp.sum(f * ar_ref[...], axis=-1, keepdims=True)
  eler_ref[...] = jnp.concatenate([el, er], axis=1)


def _tc2(h_pad, Wcat, al1, ar1):
  blk = 512
  grid = (NP // blk,)
  return pl.pallas_call(
      _tc2_body,
      grid=grid,
      in_specs=[
          pl.BlockSpec((blk, H0 * HID), lambda i: (i, 0)),
          pl.BlockSpec((H0 * HID, 2 * CLS), lambda i: (0, 0)),
          pl.BlockSpec((1, CLS), lambda i: (0, 0)),
          pl.BlockSpec((1, CLS), lambda i: (0, 0)),
      ],
      out_specs=[
          pl.BlockSpec((blk, 2 * CLS), lambda i: (i, 0)),
          pl.BlockSpec((blk, 2 * CLS), lambda i: (i, 0)),
          pl.BlockSpec((blk, 2), lambda i: (i, 0)),
      ],
      out_shape=[
          jax.ShapeDtypeStruct((NP, 2 * CLS), jnp.float32),
          jax.ShapeDtypeStruct((NP, 2 * CLS), jnp.float32),
          jax.ShapeDtypeStruct((NP, 2), jnp.float32),
      ],
  )(h_pad, Wcat, al1, ar1)


def _make_attn(H):
  HPS = max(H // NSC, 1)
  EP = E // NTL
  VE = EP // LANES
  TS = NP // NTL
  NV = NP // LANES

  def body(src_hbm, dst_hbm, eler_hbm, out_hbm,
           src_v, dst_v, e_buf, el_v, er_v, ms_v, sp_v,
           part_sh, fin_sh, sem):
    cid = lax.axis_index("c")
    sid = lax.axis_index("s")
    base_e = sid * EP
    pltpu.sync_copy(src_hbm.at[pl.ds(base_e, EP)], src_v)
    pltpu.sync_copy(dst_hbm.at[pl.ds(base_e, EP)], dst_v)

    for h in range(HPS):
      head = cid * HPS + h

      @pl.when(head < H)
      def _head():
        pltpu.sync_copy(eler_hbm.at[pl.ds(head * NP, NP)], el_v)
        pltpu.sync_copy(eler_hbm.at[pl.ds((H + head) * NP, NP)], er_v)

        @pl.loop(0, NV)
        def _initm(j):
          ms_v[pl.ds(j * 16, 16)] = jnp.full((16,), -1e30, jnp.float32)

        @pl.loop(0, VE)
        def _scan1(v):
          sv = src_v[pl.ds(v * 16, 16)]
          dv = dst_v[pl.ds(v * 16, 16)]
          e = plsc.load_gather(el_v, [sv]) + plsc.load_gather(er_v, [dv])
          e = jnp.where(e >= 0.0, e, e * NEG)
          e_buf[pl.ds(v * 16, 16)] = e
          cur = plsc.load_gather(ms_v, [dv])
          plsc.store_scatter(ms_v, [dv], jnp.maximum(cur, e))

        pltpu.sync_copy(ms_v, part_sh.at[sid])
        plsc.subcore_barrier()
        for t in range(NTL):
          pltpu.sync_copy(part_sh.at[t, pl.ds(sid * TS, TS)],
                          er_v.at[pl.ds(t * TS, TS)])

        @pl.loop(0, TS // 16)
        def _redm(j):
          acc = er_v[pl.ds(j * 16, 16)]
          for t in range(1, NTL):
            acc = jnp.maximum(acc, er_v[pl.ds(t * TS + j * 16, 16)])
          ms_v[pl.ds(j * 16, 16)] = acc

        pltpu.sync_copy(ms_v.at[pl.ds(0, TS)], fin_sh.at[pl.ds(sid * TS, TS)])
        plsc.subcore_barrier()
        pltpu.sync_copy(fin_sh, ms_v)
        plsc.subcore_barrier()

        @pl.loop(0, NV)
        def _inits(j):
          sp_v[pl.ds(j * 16, 16)] = jnp.zeros((16,), jnp.float32)

        @pl.loop(0, VE)
        def _scan2(v):
          dv = dst_v[pl.ds(v * 16, 16)]
          ee = jnp.exp(e_buf[pl.ds(v * 16, 16)] - plsc.load_gather(ms_v, [dv]))
          e_buf[pl.ds(v * 16, 16)] = ee
          plsc.addupdate_scatter(sp_v, [dv], ee)

        pltpu.sync_copy(sp_v, part_sh.at[sid])
        plsc.subcore_barrier()
        for t in range(NTL):
          pltpu.sync_copy(part_sh.at[t, pl.ds(sid * TS, TS)],
                          er_v.at[pl.ds(t * TS, TS)])

        @pl.loop(0, TS // 16)
        def _reds(j):
          acc = er_v[pl.ds(j * 16, 16)]
          for t in range(1, NTL):
            acc = acc + er_v[pl.ds(t * TS + j * 16, 16)]
          sp_v[pl.ds(j * 16, 16)] = acc

        pltpu.sync_copy(sp_v.at[pl.ds(0, TS)], fin_sh.at[pl.ds(sid * TS, TS)])
        plsc.subcore_barrier()
        pltpu.sync_copy(fin_sh, sp_v)
        plsc.subcore_barrier()

        for half in range(2):
          @pl.loop(0, VE // 2)
          def _scan3(v):
            off = half * (EP // 2) + v * 16
            dv = dst_v[pl.ds(off, 16)]
            s = plsc.load_gather(sp_v, [dv])
            el_v[pl.ds(v * 16, 16)] = e_buf[pl.ds(off, 16)] / (s + 1e-9)
          pltpu.sync_copy(
              el_v.at[pl.ds(0, EP // 2)],
              out_hbm.at[pl.ds(head * E + base_e + half * (EP // 2), EP // 2)])

  kern = pl.kernel(
      body,
      out_type=jax.ShapeDtypeStruct((H * E,), jnp.float32),
      mesh=_MESH,
      compiler_params=_SC_PARAMS,
      scratch_types=[
          pltpu.VMEM((EP,), jnp.int32),
          pltpu.VMEM((EP,), jnp.int32),
          pltpu.VMEM((EP,), jnp.float32),
          pltpu.VMEM((NP,), jnp.float32),
          pltpu.VMEM((NP,), jnp.float32),
          pltpu.VMEM((NP,), jnp.float32),
          pltpu.VMEM((NP,), jnp.float32),
          pltpu.VMEM_SHARED((NTL, NP), jnp.float32),
          pltpu.VMEM_SHARED((NP,), jnp.float32),
          pltpu.SemaphoreType.DMA,
      ],
  )
  return kern


def _make_agg(GR, H, CS, NCH, has_init, RB):
  EP = E // NTL
  VE = EP // LANES
  assert RB * GR <= 128 and RB * H <= 128
  DJ = (128 // GR) // 16 if H == 1 else 128 // 16
  RPT = CS // NTL
  FR = RPT * GR

  def body(*refs):
    if has_init:
      (src_hbm, dst_hbm, a_hbm, feat_hbm, init_hbm, out_hbm,
       src_v, dst_v, qpk, rows_v, a0q_v, gi_v, si_v, ai_v,
       zero_v, acc_sh, sem, sem2) = refs
    else:
      (src_hbm, dst_hbm, a_hbm, feat_hbm, out_hbm,
       src_v, dst_v, qpk, rows_v, a0q_v, gi_v, si_v, ai_v,
       zero_v, acc_sh, sem, sem2) = refs
    cid = lax.axis_index("c")
    sid = lax.axis_index("s")
    base_e = sid * EP
    pltpu.sync_copy(src_hbm.at[pl.ds(base_e, EP)], src_v)
    pltpu.sync_copy(dst_hbm.at[pl.ds(base_e, EP)], dst_v)

    for r in range(8):
      @pl.loop(0, 8)
      def _z(i):
        zero_v[r, pl.ds(i * 16, 16)] = jnp.zeros((16,), jnp.float32)

    iota = lax.iota(jnp.int32, 16)

    @pl.loop(0, NCH)
    def _chunk_loop(ch):
      base = ch * CS

      @pl.when((ch % NSC) == cid)
      def _chunk():
        if has_init:
          for k in range(FR // 8):
            pltpu.sync_copy(
                init_hbm.at[pl.ds((base * GR) + sid * FR + k * 8, 8)],
                acc_sh.at[pl.ds(sid * FR + k * 8, 8)])
        else:
          for k in range(FR // 8):
            pltpu.sync_copy(zero_v, acc_sh.at[pl.ds(sid * FR + k * 8, 8)])
        plsc.subcore_barrier()

        def _scan(v, qcnt):
          dv = dst_v[pl.ds(v * 16, 16)]
          msk = (dv >= base) & (dv < base + CS)
          pos = qcnt + plsc.cumsum(jnp.where(msk, 1, 0).astype(jnp.int32)) - 1
          qval = ((v * 16 + iota) << 12) | (dv - base)
          plsc.store_scatter(qpk, [pos], qval, mask=msk)
          return qcnt + plsc.all_reduce_population_count(msk)

        qcnt = lax.fori_loop(0, VE, _scan, jnp.zeros((16,), jnp.int32))
        qn = qcnt[0]
        for pv in range(RB // 16):
          plsc.store_scatter(qpk, [qn + pv * 16 + iota],
                             jnp.full((16,), CS, jnp.int32))

        nb = (qn + RB - 1) // RB

        def _batch(b, _):
          q0 = b * RB
          for pv in range(RB // 16):
            qv = qpk[pl.ds(q0 + pv * 16, 16)]
            rel = qv >> 12
            ldst = qv & 4095
            sv = plsc.load_gather(src_v, [rel])
            epos = (pv * 16 + iota) * GR
            for k in range(GR):
              plsc.store_scatter(gi_v, [epos + k], sv * GR + k)
              plsc.store_scatter(si_v, [epos + k], ldst * GR + k)
            for h in range(H):
              ai_v[pl.ds(h * RB + pv * 16, 16)] = rel + (base_e + h * E)
          cp = pltpu.async_copy(feat_hbm.at[gi_v], rows_v, sem)
          ac = pltpu.async_copy(a_hbm.at[ai_v], a0q_v, sem2)
          cp.wait()
          ac.wait()

          def _srow(r, _):
            for h in range(H):
              scale = plsc.load_gather(
                  a0q_v, [jnp.full((16,), h * RB + r, jnp.int32)])
              for j in range(DJ):
                rows_v[r * GR + h, pl.ds(j * 16, 16)] = (
                    rows_v[r * GR + h, pl.ds(j * 16, 16)] * scale)
            return 0

          lax.fori_loop(0, RB, _srow, 0)
          pltpu.sync_copy(rows_v, acc_sh.at[si_v], add=True)
          return 0

        lax.fori_loop(0, nb, _batch, 0)
        plsc.subcore_barrier()

        for k in range(FR // 16):
          pltpu.sync_copy(acc_sh.at[pl.ds(sid * FR + k * 16, 16)],
                          out_hbm.at[pl.ds(base * GR + sid * FR + k * 16, 16)])
        plsc.subcore_barrier()

  scratch = [
      pltpu.VMEM((EP,), jnp.int32),
      pltpu.VMEM((EP,), jnp.int32),
      pltpu.VMEM((EP + RB,), jnp.int32),
      pltpu.VMEM((RB * GR, 128), jnp.float32),
      pltpu.VMEM((H * RB,), jnp.float32),
      pltpu.VMEM((RB * GR,), jnp.int32),
      pltpu.VMEM((RB * GR,), jnp.int32),
      pltpu.VMEM((H * RB,), jnp.int32),
      pltpu.VMEM((8, 128), jnp.float32),
      pltpu.VMEM_SHARED((CS * GR + 64, 128), jnp.float32),
      pltpu.SemaphoreType.DMA,
      pltpu.SemaphoreType.DMA,
  ]
  kern = pl.kernel(
      body,
      out_type=jax.ShapeDtypeStruct((NP * GR, 128), jnp.float32),
      mesh=_MESH,
      compiler_params=_SC_PARAMS,
      scratch_types=scratch,
  )
  return kern


_ATTN0 = _make_attn(H0)
_ATTN1 = _make_attn(H1)
_AGG0 = _make_agg(H0, H0, 640, NP // 640, False, 16)
_AGG1 = _make_agg(1, H1, 2560, NP // 2560, True, 32)


def kernel(inputs, edge_index, W0, al0, ar0, W1, al1, ar1, resW1):
  src = edge_index[0]
  dst = edge_index[1]
  x_pad = jnp.pad(inputs, ((0, NP - N), (0, 0)))

  feat0, eler0 = _tc1(x_pad, W0, al0, ar0)
  elerT0 = eler0.T.reshape(-1)
  a0_flat = _ATTN0(src, dst, elerT0)
  h_prer = _AGG0(src, dst, a0_flat, feat0.reshape(NP * H0, HID))
  h_pre = h_prer.reshape(NP, H0 * HID)

  Wcat = jnp.concatenate([W1, resW1], axis=1)
  feat1, res1, eler1 = _tc2(h_pre, Wcat, al1, ar1)
  elerT1 = eler1.T.reshape(-1)
  a1_flat = _ATTN1(src, dst, elerT1)
  out1 = _AGG1(src, dst, a1_flat, feat1, res1)

  logits = out1[:N, :CLS]
  a0 = a0_flat.reshape(H0, E).T
  a1 = a1_flat.reshape(E, H1)
  return logits, a0, a1
